# baseline (speedup 1.0000x reference)
import functools

import jax
import jax.numpy as jnp
from jax import lax
from jax.experimental import pallas as pl
from jax.experimental.pallas import tpu as pltpu, tpu_sc as plsc

N = 10000
E = 160000
D = 256
ED = 16
H = 256

BN = 1000
BE = 2000


def _silu(v):
    return v * jax.nn.sigmoid(v)


def _node_proj_body(x_ref, w_ref, a_ref, b_ref):
    xb = x_ref[...]
    a_ref[...] = jnp.dot(xb, w_ref[:D, :], preferred_element_type=jnp.float32)
    b_ref[...] = jnp.dot(xb, w_ref[D:, :], preferred_element_type=jnp.float32)


def _node_proj(x, w_ab):
    return pl.pallas_call(
        _node_proj_body,
        grid=(N // BN,),
        in_specs=[
            pl.BlockSpec((BN, D), lambda i: (i, 0)),
            pl.BlockSpec((2 * D, H), lambda i: (0, 0)),
        ],
        out_specs=[
            pl.BlockSpec((BN, H), lambda i: (i, 0)),
            pl.BlockSpec((BN, H), lambda i: (i, 0)),
        ],
        out_shape=[
            jax.ShapeDtypeStruct((N, H), jnp.float32),
            jax.ShapeDtypeStruct((N, H), jnp.float32),
        ],
    )(x, w_ab)


def _edge_mlp_body(g_ref, d2_ref, ea_ref, wd_ref, we_ref, b1_ref, w2_ref,
                   b2_ref, v_ref):
    u = (g_ref[...]
         + d2_ref[...] * wd_ref[...]
         + jnp.dot(ea_ref[...], we_ref[...], preferred_element_type=jnp.float32)
         + b1_ref[...])
    m1 = _silu(u)
    v = jnp.dot(m1, w2_ref[...], preferred_element_type=jnp.float32) + b2_ref[...]
    v_ref[...] = _silu(v)


def _edge_mlp(g, d2, ea, w_d, w_e, b1, w2, b2):
    return pl.pallas_call(
        _edge_mlp_body,
        grid=(E // BE,),
        in_specs=[
            pl.BlockSpec((BE, H), lambda i: (i, 0)),
            pl.BlockSpec((BE, 1), lambda i: (i, 0)),
            pl.BlockSpec((BE, ED), lambda i: (i, 0)),
            pl.BlockSpec((1, H), lambda i: (0, 0)),
            pl.BlockSpec((ED, H), lambda i: (0, 0)),
            pl.BlockSpec((1, H), lambda i: (0, 0)),
            pl.BlockSpec((H, H), lambda i: (0, 0)),
            pl.BlockSpec((1, H), lambda i: (0, 0)),
        ],
        out_specs=pl.BlockSpec((BE, H), lambda i: (i, 0)),
        out_shape=jax.ShapeDtypeStruct((E, H), jnp.float32),
    )(g, d2, ea, w_d, w_e, b1, w2, b2)


def _node_mlp_body(x_ref, s_ref, cnt_ref, w3_ref, b3_ref, w4_ref, b4_ref,
                   h_ref, *, residual):
    xb = x_ref[...]
    agg = s_ref[...] / jnp.maximum(cnt_ref[...], 1.0)
    pre = (jnp.dot(xb, w3_ref[:D, :], preferred_element_type=jnp.float32)
           + jnp.dot(agg, w3_ref[D:, :], preferred_element_type=jnp.float32)
           + b3_ref[...])
    h = jnp.dot(_silu(pre), w4_ref[...], preferred_element_type=jnp.float32) \
        + b4_ref[...]
    if residual:
        h = h + xb
    h_ref[...] = h


def _node_mlp(x, s, cnt, w3, b3, w4, b4, residual):
    return pl.pallas_call(
        functools.partial(_node_mlp_body, residual=residual),
        grid=(N // BN,),
        in_specs=[
            pl.BlockSpec((BN, D), lambda i: (i, 0)),
            pl.BlockSpec((BN, H), lambda i: (i, 0)),
            pl.BlockSpec((BN, 1), lambda i: (i, 0)),
            pl.BlockSpec((D + H, H), lambda i: (0, 0)),
            pl.BlockSpec((1, H), lambda i: (0, 0)),
            pl.BlockSpec((H, H), lambda i: (0, 0)),
            pl.BlockSpec((1, H), lambda i: (0, 0)),
        ],
        out_specs=pl.BlockSpec((BN, H), lambda i: (i, 0)),
        out_shape=jax.ShapeDtypeStruct((N, H), jnp.float32),
    )(x, s, cnt, w3, b3, w4, b4)


def _conv(x, src, dst, d2, ea, cnt, W1, b1, W2, b2, W3, b3, W4, b4, residual):
    w_ab = W1[:2 * D, :]
    a, b = _node_proj(x, w_ab)
    g = a[dst] + b[src]
    w_d = W1[2 * D:2 * D + 1, :]
    w_e = W1[2 * D + 1:, :]
    v = _edge_mlp(g, d2, ea, w_d, w_e, b1.reshape(1, H), W2,
                  b2.reshape(1, H))
    s = jax.ops.segment_sum(v, dst, num_segments=N)
    return _node_mlp(x, s, cnt, W3, b3.reshape(1, H), W4, b4.reshape(1, H),
                     residual)


def kernel(x, edge_index, edge_attr, pos,
           c1_W1, c1_b1, c1_W2, c1_b2, c1_W3, c1_b3, c1_W4, c1_b4,
           c2_W1, c2_b1, c2_W2, c2_b2, c2_W3, c2_b3, c2_W4, c2_b4):
    src = edge_index[0]
    dst = edge_index[1]
    rel = pos[dst] - pos[src]
    d2 = jnp.sum(rel * rel, axis=-1, keepdims=True)
    cnt = jax.ops.segment_sum(jnp.ones((E, 1), jnp.float32), dst,
                              num_segments=N)
    h = _conv(x, src, dst, d2, edge_attr, cnt,
              c1_W1, c1_b1, c1_W2, c1_b2, c1_W3, c1_b3, c1_W4, c1_b4, True)
    h = _conv(h, src, dst, d2, edge_attr, cnt,
              c2_W1, c2_b1, c2_W2, c2_b2, c2_W3, c2_b3, c2_W4, c2_b4, False)
    return h


# speedup vs baseline: 1.8533x; 1.8533x over previous
import functools

import jax
import jax.numpy as jnp
from jax import lax
from jax.experimental import pallas as pl
from jax.experimental.pallas import tpu as pltpu, tpu_sc as plsc

N = 10000
E = 160000
D = 256
ED = 16
H = 256

BN = 1000
BE = 2000


def _silu(v):
    return v * jax.nn.sigmoid(v)


def _node_proj_body(x_ref, w_ref, a_ref, b_ref):
    xb = x_ref[...]
    a_ref[...] = jnp.dot(xb, w_ref[:D, :], preferred_element_type=jnp.float32)
    b_ref[...] = jnp.dot(xb, w_ref[D:, :], preferred_element_type=jnp.float32)


def _node_proj(x, w_ab):
    return pl.pallas_call(
        _node_proj_body,
        grid=(N // BN,),
        in_specs=[
            pl.BlockSpec((BN, D), lambda i: (i, 0)),
            pl.BlockSpec((2 * D, H), lambda i: (0, 0)),
        ],
        out_specs=[
            pl.BlockSpec((BN, H), lambda i: (i, 0)),
            pl.BlockSpec((BN, H), lambda i: (i, 0)),
        ],
        out_shape=[
            jax.ShapeDtypeStruct((N, H), jnp.float32),
            jax.ShapeDtypeStruct((N, H), jnp.float32),
        ],
    )(x, w_ab)


def _edge_mlp_body(g_ref, d2_ref, ea_ref, wd_ref, we_ref, b1_ref, w2_ref,
                   b2_ref, v_ref):
    u = (g_ref[...]
         + d2_ref[...] * wd_ref[...]
         + jnp.dot(ea_ref[...], we_ref[...], preferred_element_type=jnp.float32)
         + b1_ref[...])
    m1 = _silu(u)
    v = jnp.dot(m1, w2_ref[...], preferred_element_type=jnp.float32) + b2_ref[...]
    v_ref[...] = _silu(v)


def _edge_mlp(g, d2, ea, w_d, w_e, b1, w2, b2):
    return pl.pallas_call(
        _edge_mlp_body,
        grid=(E // BE,),
        in_specs=[
            pl.BlockSpec((BE, H), lambda i: (i, 0)),
            pl.BlockSpec((BE, 1), lambda i: (i, 0)),
            pl.BlockSpec((BE, ED), lambda i: (i, 0)),
            pl.BlockSpec((1, H), lambda i: (0, 0)),
            pl.BlockSpec((ED, H), lambda i: (0, 0)),
            pl.BlockSpec((1, H), lambda i: (0, 0)),
            pl.BlockSpec((H, H), lambda i: (0, 0)),
            pl.BlockSpec((1, H), lambda i: (0, 0)),
        ],
        out_specs=pl.BlockSpec((BE, H), lambda i: (i, 0)),
        out_shape=jax.ShapeDtypeStruct((E, H), jnp.float32),
    )(g, d2, ea, w_d, w_e, b1, w2, b2)


def _node_mlp_body(x_ref, s_ref, cnt_ref, w3_ref, b3_ref, w4_ref, b4_ref,
                   h_ref, *, residual):
    xb = x_ref[...]
    agg = s_ref[...] / jnp.maximum(cnt_ref[...], 1.0)
    pre = (jnp.dot(xb, w3_ref[:D, :], preferred_element_type=jnp.float32)
           + jnp.dot(agg, w3_ref[D:, :], preferred_element_type=jnp.float32)
           + b3_ref[...])
    h = jnp.dot(_silu(pre), w4_ref[...], preferred_element_type=jnp.float32) \
        + b4_ref[...]
    if residual:
        h = h + xb
    h_ref[...] = h


def _node_mlp(x, s, cnt, w3, b3, w4, b4, residual):
    return pl.pallas_call(
        functools.partial(_node_mlp_body, residual=residual),
        grid=(N // BN,),
        in_specs=[
            pl.BlockSpec((BN, D), lambda i: (i, 0)),
            pl.BlockSpec((BN, H), lambda i: (i, 0)),
            pl.BlockSpec((BN, 1), lambda i: (i, 0)),
            pl.BlockSpec((D + H, H), lambda i: (0, 0)),
            pl.BlockSpec((1, H), lambda i: (0, 0)),
            pl.BlockSpec((H, H), lambda i: (0, 0)),
            pl.BlockSpec((1, H), lambda i: (0, 0)),
        ],
        out_specs=pl.BlockSpec((BN, H), lambda i: (i, 0)),
        out_shape=jax.ShapeDtypeStruct((N, H), jnp.float32),
    )(x, s, cnt, w3, b3, w4, b4)


_NC = 2
_NS = 16
_NW = _NC * _NS
_C = 128
_NCHUNK = E // _C
_GIT = (_NCHUNK + _NW - 1) // _NW
_SIT = (_NCHUNK + _NS - 1) // _NS
_HALF = N // 2
_RPS = 320
_ACC_R = _RPS * _NS + 8
_TRASH = 5120
_TAIL = _HALF - (_NS - 1) * _RPS


def _sc_mesh():
    return plsc.VectorSubcoreMesh(core_axis_name="c", subcore_axis_name="s")


def _gather_body_common(a_hbm, b_hbm, dst_hbm, src_hbm, g_hbm,
                        bufa, bufb, dstb, srcb, sem1, sem2,
                        extra=None):
    wid = lax.axis_index("s") * _NC + lax.axis_index("c")

    if extra is not None:
        px_hbm, py_hbm, pz_hbm, d2_hbm, pxv, pyv, pzv, d2b = extra
        pltpu.sync_copy(px_hbm, pxv)
        pltpu.sync_copy(py_hbm, pyv)
        pltpu.sync_copy(pz_hbm, pzv)

    def chunk_step(i, _):
        chunk = wid + _NW * i

        @pl.when(chunk < _NCHUNK)
        def _():
            base = chunk * _C
            pltpu.sync_copy(dst_hbm.at[pl.ds(base, _C)], dstb)
            pltpu.sync_copy(src_hbm.at[pl.ds(base, _C)], srcb)
            cp1 = pltpu.async_copy(a_hbm.at[dstb], bufa, sem1)
            cp2 = pltpu.async_copy(b_hbm.at[srcb], bufb, sem2)
            cp1.wait()
            cp2.wait()

            def add_row(r, _):
                for j in range(H // 16):
                    sl = pl.ds(j * 16, 16)
                    bufa[r, sl] = bufa[r, sl] + bufb[r, sl]
                return 0

            lax.fori_loop(0, _C, add_row, 0, unroll=False)

            if extra is not None:
                for j in range(_C // 16):
                    sl = pl.ds(j * 16, 16)
                    dv = dstb[sl]
                    sv = srcb[sl]
                    dx = plsc.load_gather(pxv, [dv]) - plsc.load_gather(pxv, [sv])
                    dy = plsc.load_gather(pyv, [dv]) - plsc.load_gather(pyv, [sv])
                    dz = plsc.load_gather(pzv, [dv]) - plsc.load_gather(pzv, [sv])
                    d2b[sl] = dx * dx + dy * dy + dz * dz
                pltpu.sync_copy(d2b, d2_hbm.at[pl.ds(base, _C)])

            pltpu.sync_copy(bufa, g_hbm.at[pl.ds(base, _C)])

        return 0

    lax.fori_loop(0, _GIT, chunk_step, 0, unroll=False)


def _gather_d2_kernel(a, b, dst, src, px, py, pz):
    def body(a_hbm, b_hbm, dst_hbm, src_hbm, px_hbm, py_hbm, pz_hbm,
             g_hbm, d2_hbm, bufa, bufb, dstb, srcb, pxv, pyv, pzv, d2b,
             sem1, sem2):
        _gather_body_common(
            a_hbm, b_hbm, dst_hbm, src_hbm, g_hbm,
            bufa, bufb, dstb, srcb, sem1, sem2,
            extra=(px_hbm, py_hbm, pz_hbm, d2_hbm, pxv, pyv, pzv, d2b))

    return pl.kernel(
        body,
        out_type=[
            jax.ShapeDtypeStruct((E, H), jnp.float32),
            jax.ShapeDtypeStruct((E,), jnp.float32),
        ],
        mesh=_sc_mesh(),
        compiler_params=pltpu.CompilerParams(needs_layout_passes=False),
        scratch_types=[
            pltpu.VMEM((_C, H), jnp.float32),
            pltpu.VMEM((_C, H), jnp.float32),
            pltpu.VMEM((_C,), jnp.int32),
            pltpu.VMEM((_C,), jnp.int32),
            pltpu.VMEM((N,), jnp.float32),
            pltpu.VMEM((N,), jnp.float32),
            pltpu.VMEM((N,), jnp.float32),
            pltpu.VMEM((_C,), jnp.float32),
            pltpu.SemaphoreType.DMA,
            pltpu.SemaphoreType.DMA,
        ],
    )(a, b, dst, src, px, py, pz)


def _gather_kernel(a, b, dst, src):
    def body(a_hbm, b_hbm, dst_hbm, src_hbm, g_hbm,
             bufa, bufb, dstb, srcb, sem1, sem2):
        _gather_body_common(a_hbm, b_hbm, dst_hbm, src_hbm, g_hbm,
                            bufa, bufb, dstb, srcb, sem1, sem2)

    return pl.kernel(
        body,
        out_type=jax.ShapeDtypeStruct((E, H), jnp.float32),
        mesh=_sc_mesh(),
        compiler_params=pltpu.CompilerParams(needs_layout_passes=False),
        scratch_types=[
            pltpu.VMEM((_C, H), jnp.float32),
            pltpu.VMEM((_C, H), jnp.float32),
            pltpu.VMEM((_C,), jnp.int32),
            pltpu.VMEM((_C,), jnp.int32),
            pltpu.SemaphoreType.DMA,
            pltpu.SemaphoreType.DMA,
        ],
    )(a, b, dst, src)


_RPT = 320
_ACCR = _RPT + 8
_LTRASH = _RPT
_CAP = 8192
_SB = 64
_SCN = 2048


def _scatter_body_common(v_hbm, dst_hbm, s_hbm, acc, vb, dstb, listE, listL,
                         cnt=None):
    wid = lax.axis_index("s") * _NC + lax.axis_index("c")
    base = wid * _RPT
    if cnt is not None:
        cnt_hbm, cntacc = cnt

    zvec = jnp.zeros((16,), jnp.float32)
    zi = jnp.zeros((16,), jnp.int32)
    ti = jnp.full((16,), _LTRASH, jnp.int32)
    ovec = jnp.ones((16,), jnp.float32)

    def zero_row(r, _):
        for j in range(H // 16):
            acc[r, pl.ds(j * 16, 16)] = zvec
        if cnt is not None:
            cntacc[r, :] = zvec
        return 0

    lax.fori_loop(0, _ACCR, zero_row, 0, unroll=False)

    def prefill(i, _):
        listE[pl.ds(i * 16, 16)] = wid * 4096 + i * 16 + lax.iota(jnp.int32, 16)
        listL[pl.ds(i * 16, 16)] = ti
        return 0

    lax.fori_loop(0, _CAP // 16, prefill, 0, unroll=False)

    def scan_chunk(ci, off):
        pltpu.sync_copy(dst_hbm.at[pl.ds(ci * _SCN, _SCN)], dstb)

        def step(j, off):
            dv = dstb[pl.ds(j * 16, 16)]
            inb = (dv >= base) & (dv < base + _RPT)
            eid = ci * _SCN + j * 16 + lax.iota(jnp.int32, 16)
            lid = dv - base

            @pl.when(off <= _CAP - 16)
            def _():
                plsc.store_compressed(listE.at[pl.ds(off, 16)], eid,
                                      mask=inb)
                plsc.store_compressed(listL.at[pl.ds(off, 16)], lid,
                                      mask=inb)

            return off + jnp.sum(inb.astype(jnp.int32))

        return lax.fori_loop(0, _SCN // 16, step, off, unroll=False)

    off = lax.fori_loop(0, E // _SCN, scan_chunk, 0, unroll=False)

    def add_row(li, r):
        for j in range(H // 16):
            sl = pl.ds(j * 16, 16)
            acc[li, sl] = acc[li, sl] + vb[r, sl]
        if cnt is not None:
            cntacc[li, pl.ds(0, 16)] = cntacc[li, pl.ds(0, 16)] + ovec

    @pl.when(off <= _CAP)
    def _():
        nbat = (off + _SB - 1) // _SB

        def batch(b, _):
            @pl.when(b < nbat)
            def _():
                pltpu.sync_copy(v_hbm.at[listE.at[pl.ds(b * _SB, _SB)]], vb)

                def addgrp(g, _):
                    lv = listL[pl.ds(b * _SB + g * 16, 16)]
                    for k in range(16):
                        add_row(lv[k], g * 16 + k)
                    return 0

                lax.fori_loop(0, _SB // 16, addgrp, 0, unroll=False)

            return 0

        lax.fori_loop(0, _CAP // _SB, batch, 0, unroll=False)

    @pl.when(off > _CAP)
    def _():
        def chunk(ci, _):
            pltpu.sync_copy(dst_hbm.at[pl.ds(ci * _SB, _SB)],
                            dstb.at[pl.ds(0, _SB)])
            pltpu.sync_copy(v_hbm.at[pl.ds(ci * _SB, _SB)], vb)

            def grp(g, _):
                dv = dstb[pl.ds(g * 16, 16)]
                lid = dv - base
                for k in range(16):
                    d = dv[k]

                    @pl.when((d >= base) & (d < base + _RPT))
                    def _():
                        add_row(lid[k], g * 16 + k)
                return 0

            lax.fori_loop(0, _SB // 16, grp, 0, unroll=False)
            return 0

        lax.fori_loop(0, E // _SB, chunk, 0, unroll=False)

    @pl.when(wid < _NW - 1)
    def _():
        pltpu.sync_copy(acc.at[pl.ds(0, _RPT)], s_hbm.at[pl.ds(base, _RPT)])
        if cnt is not None:
            pltpu.sync_copy(cntacc.at[pl.ds(0, _RPT)],
                            cnt_hbm.at[pl.ds(base, _RPT)])

    tail = N - (_NW - 1) * _RPT

    @pl.when(wid == _NW - 1)
    def _():
        pltpu.sync_copy(acc.at[pl.ds(0, tail)], s_hbm.at[pl.ds(base, tail)])
        if cnt is not None:
            pltpu.sync_copy(cntacc.at[pl.ds(0, tail)],
                            cnt_hbm.at[pl.ds(base, tail)])


def _count_kernel(dst):
    def body(dst_hbm, cnt_hbm, cntacc, dstb, listL):
        wid = lax.axis_index("s") * _NC + lax.axis_index("c")
        base = wid * _RPT
        zvec = jnp.zeros((16,), jnp.float32)
        ovec = jnp.ones((16,), jnp.float32)
        ti = jnp.full((16,), _LTRASH, jnp.int32)

        def zero_row(r, _):
            cntacc[r, pl.ds(0, 16)] = zvec
            return 0

        lax.fori_loop(0, _ACCR, zero_row, 0, unroll=False)

        def prefill(i, _):
            listL[pl.ds(i * 16, 16)] = ti
            return 0

        lax.fori_loop(0, _CAP // 16, prefill, 0, unroll=False)

        def scan_chunk(ci, off):
            pltpu.sync_copy(dst_hbm.at[pl.ds(ci * _SCN, _SCN)], dstb)

            def step(j, off):
                dv = dstb[pl.ds(j * 16, 16)]
                inb = (dv >= base) & (dv < base + _RPT)
                lid = dv - base

                @pl.when(off <= _CAP - 16)
                def _():
                    plsc.store_compressed(listL.at[pl.ds(off, 16)], lid,
                                          mask=inb)

                return off + jnp.sum(inb.astype(jnp.int32))

            return lax.fori_loop(0, _SCN // 16, step, off, unroll=False)

        off = lax.fori_loop(0, E // _SCN, scan_chunk, 0, unroll=False)

        @pl.when(off <= _CAP)
        def _():
            ngrp = (off + 15) // 16

            def grp(g, _):
                @pl.when(g < ngrp)
                def _():
                    lv = listL[pl.ds(g * 16, 16)]
                    for k in range(16):
                        li = lv[k]
                        cntacc[li, pl.ds(0, 16)] = cntacc[li, pl.ds(0, 16)] + ovec
                return 0

            lax.fori_loop(0, _CAP // 16, grp, 0, unroll=False)

        @pl.when(off > _CAP)
        def _():
            def chunk(ci, _):
                pltpu.sync_copy(dst_hbm.at[pl.ds(ci * _SCN, _SCN)], dstb)

                def sgrp(j, _):
                    dv = dstb[pl.ds(j * 16, 16)]
                    lid = dv - base
                    for k in range(16):
                        d = dv[k]

                        @pl.when((d >= base) & (d < base + _RPT))
                        def _():
                            li = lid[k]
                            cntacc[li, pl.ds(0, 16)] = cntacc[li, pl.ds(0, 16)] + ovec
                    return 0

                lax.fori_loop(0, _SCN // 16, sgrp, 0, unroll=False)
                return 0

            lax.fori_loop(0, E // _SCN, chunk, 0, unroll=False)

        @pl.when(wid < _NW - 1)
        def _():
            pltpu.sync_copy(cntacc.at[pl.ds(0, _RPT)],
                            cnt_hbm.at[pl.ds(base, _RPT)])

        tail = N - (_NW - 1) * _RPT

        @pl.when(wid == _NW - 1)
        def _():
            pltpu.sync_copy(cntacc.at[pl.ds(0, tail)],
                            cnt_hbm.at[pl.ds(base, tail)])

    return pl.kernel(
        body,
        out_type=jax.ShapeDtypeStruct((N, 16), jnp.float32),
        mesh=_sc_mesh(),
        compiler_params=pltpu.CompilerParams(needs_layout_passes=False),
        scratch_types=[
            pltpu.VMEM((_ACCR, 16), jnp.float32),
            pltpu.VMEM((_SCN,), jnp.int32),
            pltpu.VMEM((_CAP,), jnp.int32),
        ],
    )(dst)


def _scatter_kernel(v, dst):
    def body(v_hbm, dst_hbm, s_hbm, acc, vb, dstb, listE, listL):
        _scatter_body_common(v_hbm, dst_hbm, s_hbm, acc, vb, dstb, listE,
                             listL)

    return pl.kernel(
        body,
        out_type=jax.ShapeDtypeStruct((N, H), jnp.float32),
        mesh=_sc_mesh(),
        compiler_params=pltpu.CompilerParams(needs_layout_passes=False),
        scratch_types=[
            pltpu.VMEM((_ACCR, H), jnp.float32),
            pltpu.VMEM((_SB, H), jnp.float32),
            pltpu.VMEM((_SCN,), jnp.int32),
            pltpu.VMEM((_CAP,), jnp.int32),
            pltpu.VMEM((_CAP,), jnp.int32),
        ],
    )(v, dst)


def _conv(x, src, dst, d2, ea, cnt, pxyz, W1, b1, W2, b2, W3, b3, W4, b4,
          residual):
    first = d2 is None
    a, b = _node_proj(x, W1[:2 * D, :])
    if first:
        g, d2f = _gather_d2_kernel(a, b, dst, src, *pxyz)
        d2 = d2f.reshape(E, 1)
    else:
        g = _gather_kernel(a, b, dst, src)
    w_d = W1[2 * D:2 * D + 1, :]
    w_e = W1[2 * D + 1:, :]
    v = _edge_mlp(g, d2, ea, w_d, w_e, b1.reshape(1, H), W2,
                  b2.reshape(1, H))
    s = _scatter_kernel(v, dst)
    h = _node_mlp(x, s, cnt, W3, b3.reshape(1, H), W4, b4.reshape(1, H),
                  residual)
    return h, d2


def kernel(x, edge_index, edge_attr, pos,
           c1_W1, c1_b1, c1_W2, c1_b2, c1_W3, c1_b3, c1_W4, c1_b4,
           c2_W1, c2_b1, c2_W2, c2_b2, c2_W3, c2_b3, c2_W4, c2_b4):
    src = edge_index[0]
    dst = edge_index[1]
    pxyz = (pos[:, 0], pos[:, 1], pos[:, 2])
    cnt = _count_kernel(dst)[:, :1]
    h, d2 = _conv(x, src, dst, None, edge_attr, cnt, pxyz,
                  c1_W1, c1_b1, c1_W2, c1_b2, c1_W3, c1_b3, c1_W4,
                  c1_b4, True)
    h, _ = _conv(h, src, dst, d2, edge_attr, cnt, pxyz,
                 c2_W1, c2_b1, c2_W2, c2_b2, c2_W3, c2_b3, c2_W4,
                 c2_b4, False)
    return h


# speedup vs baseline: 2.2463x; 1.2120x over previous
import functools

import jax
import jax.numpy as jnp
from jax import lax
from jax.experimental import pallas as pl
from jax.experimental.pallas import tpu as pltpu, tpu_sc as plsc

N = 10000
E = 160000
D = 256
ED = 16
H = 256

BN = 1000
BE = 2000


def _silu(v):
    return v * jax.nn.sigmoid(v)


def _node_proj_body(x_ref, w_ref, a_ref, b_ref):
    xb = x_ref[...]
    a_ref[...] = jnp.dot(xb, w_ref[:D, :], preferred_element_type=jnp.float32)
    b_ref[...] = jnp.dot(xb, w_ref[D:, :], preferred_element_type=jnp.float32)


def _node_proj(x, w_ab):
    return pl.pallas_call(
        _node_proj_body,
        grid=(N // BN,),
        in_specs=[
            pl.BlockSpec((BN, D), lambda i: (i, 0)),
            pl.BlockSpec((2 * D, H), lambda i: (0, 0)),
        ],
        out_specs=[
            pl.BlockSpec((BN, H), lambda i: (i, 0)),
            pl.BlockSpec((BN, H), lambda i: (i, 0)),
        ],
        out_shape=[
            jax.ShapeDtypeStruct((N, H), jnp.float32),
            jax.ShapeDtypeStruct((N, H), jnp.float32),
        ],
    )(x, w_ab)


def _edge_mlp_body(g_ref, d2_ref, ea_ref, wd_ref, we_ref, b1_ref, w2_ref,
                   b2_ref, v_ref):
    u = (g_ref[...]
         + d2_ref[...] * wd_ref[...]
         + jnp.dot(ea_ref[...], we_ref[...], preferred_element_type=jnp.float32)
         + b1_ref[...])
    m1 = _silu(u)
    v = jnp.dot(m1, w2_ref[...], preferred_element_type=jnp.float32) + b2_ref[...]
    v_ref[...] = _silu(v)


def _edge_mlp(g, d2, ea, w_d, w_e, b1, w2, b2):
    return pl.pallas_call(
        _edge_mlp_body,
        grid=(E // BE,),
        in_specs=[
            pl.BlockSpec((BE, H), lambda i: (i, 0)),
            pl.BlockSpec((BE, 1), lambda i: (i, 0)),
            pl.BlockSpec((BE, ED), lambda i: (i, 0)),
            pl.BlockSpec((1, H), lambda i: (0, 0)),
            pl.BlockSpec((ED, H), lambda i: (0, 0)),
            pl.BlockSpec((1, H), lambda i: (0, 0)),
            pl.BlockSpec((H, H), lambda i: (0, 0)),
            pl.BlockSpec((1, H), lambda i: (0, 0)),
        ],
        out_specs=pl.BlockSpec((BE, H), lambda i: (i, 0)),
        out_shape=jax.ShapeDtypeStruct((E, H), jnp.float32),
    )(g, d2, ea, w_d, w_e, b1, w2, b2)


def _node_mlp_body(x_ref, s_ref, cnt_ref, w3_ref, b3_ref, w4_ref, b4_ref,
                   h_ref, *, residual):
    xb = x_ref[...]
    agg = s_ref[...] / jnp.maximum(cnt_ref[...], 1.0)
    pre = (jnp.dot(xb, w3_ref[:D, :], preferred_element_type=jnp.float32)
           + jnp.dot(agg, w3_ref[D:, :], preferred_element_type=jnp.float32)
           + b3_ref[...])
    h = jnp.dot(_silu(pre), w4_ref[...], preferred_element_type=jnp.float32) \
        + b4_ref[...]
    if residual:
        h = h + xb
    h_ref[...] = h


def _node_mlp(x, s, cnt, w3, b3, w4, b4, residual):
    return pl.pallas_call(
        functools.partial(_node_mlp_body, residual=residual),
        grid=(N // BN,),
        in_specs=[
            pl.BlockSpec((BN, D), lambda i: (i, 0)),
            pl.BlockSpec((BN, H), lambda i: (i, 0)),
            pl.BlockSpec((BN, 1), lambda i: (i, 0)),
            pl.BlockSpec((D + H, H), lambda i: (0, 0)),
            pl.BlockSpec((1, H), lambda i: (0, 0)),
            pl.BlockSpec((H, H), lambda i: (0, 0)),
            pl.BlockSpec((1, H), lambda i: (0, 0)),
        ],
        out_specs=pl.BlockSpec((BN, H), lambda i: (i, 0)),
        out_shape=jax.ShapeDtypeStruct((N, H), jnp.float32),
    )(x, s, cnt, w3, b3, w4, b4)


_NC = 2
_NS = 16
_NW = _NC * _NS
_C = 128
_NCHUNK = E // _C
_GIT = (_NCHUNK + _NW - 1) // _NW
_SIT = (_NCHUNK + _NS - 1) // _NS
_HALF = N // 2
_RPS = 320
_ACC_R = _RPS * _NS + 8
_TRASH = 5120
_TAIL = _HALF - (_NS - 1) * _RPS


def _sc_mesh():
    return plsc.VectorSubcoreMesh(core_axis_name="c", subcore_axis_name="s")


_EPT = E // _NW
_GC = 64
_GFULL = _EPT // _GC
_GTAIL = _EPT - _GFULL * _GC


def _gather_body_common(a_hbm, b_hbm, dst_hbm, src_hbm, g_hbm,
                        bufa0, bufb0, bufa1, bufb1, dstb, srcb,
                        sa0, sb0, sa1, sb1, extra=None):
    wid = lax.axis_index("s") * _NC + lax.axis_index("c")
    ebase = wid * _EPT
    pltpu.sync_copy(dst_hbm.at[pl.ds(ebase, _EPT)], dstb)
    pltpu.sync_copy(src_hbm.at[pl.ds(ebase, _EPT)], srcb)

    if extra is not None:
        px_hbm, py_hbm, pz_hbm, d2_hbm, pxv, pyv, pzv, d2b = extra
        pltpu.sync_copy(px_hbm, pxv)
        pltpu.sync_copy(py_hbm, pyv)
        pltpu.sync_copy(pz_hbm, pzv)

        def d2_step(j, _):
            sl = pl.ds(j * 16, 16)
            dv = dstb[sl]
            sv = srcb[sl]
            dx = plsc.load_gather(pxv, [dv]) - plsc.load_gather(pxv, [sv])
            dy = plsc.load_gather(pyv, [dv]) - plsc.load_gather(pyv, [sv])
            dz = plsc.load_gather(pzv, [dv]) - plsc.load_gather(pzv, [sv])
            d2b[sl] = dx * dx + dy * dy + dz * dz
            return 0

        lax.fori_loop(0, _EPT // 16, d2_step, 0, unroll=False)
        sl = pl.ds(_EPT - 16, 16)
        dv = dstb[sl]
        sv = srcb[sl]
        dx = plsc.load_gather(pxv, [dv]) - plsc.load_gather(pxv, [sv])
        dy = plsc.load_gather(pyv, [dv]) - plsc.load_gather(pyv, [sv])
        dz = plsc.load_gather(pzv, [dv]) - plsc.load_gather(pzv, [sv])
        d2b[sl] = dx * dx + dy * dy + dz * dz
        pltpu.sync_copy(d2b, d2_hbm.at[pl.ds(ebase, _EPT)])

    def start(k, ba, bb, s1, s2):
        @pl.when(k < _GFULL)
        def _():
            pltpu.async_copy(a_hbm.at[dstb.at[pl.ds(k * _GC, _GC)]], ba, s1)
            pltpu.async_copy(b_hbm.at[srcb.at[pl.ds(k * _GC, _GC)]], bb, s2)

    def finish(k, ba, bb, s1, s2):
        @pl.when(k < _GFULL)
        def _():
            pltpu.make_async_copy(
                a_hbm.at[dstb.at[pl.ds(k * _GC, _GC)]], ba, s1).wait()
            pltpu.make_async_copy(
                b_hbm.at[srcb.at[pl.ds(k * _GC, _GC)]], bb, s2).wait()

            def add_row(r, _):
                for j in range(H // 16):
                    sl = pl.ds(j * 16, 16)
                    ba[r, sl] = ba[r, sl] + bb[r, sl]
                return 0

            lax.fori_loop(0, _GC, add_row, 0, unroll=False)
            pltpu.sync_copy(ba, g_hbm.at[pl.ds(ebase + k * _GC, _GC)])

    start(0, bufa0, bufb0, sa0, sb0)

    def pair(i, _):
        k = 2 * i
        start(k + 1, bufa1, bufb1, sa1, sb1)
        finish(k, bufa0, bufb0, sa0, sb0)
        start(k + 2, bufa0, bufb0, sa0, sb0)
        finish(k + 1, bufa1, bufb1, sa1, sb1)
        return 0

    lax.fori_loop(0, (_GFULL + 1) // 2, pair, 0, unroll=False)

    t0 = _GFULL * _GC
    ta = bufa0.at[pl.ds(0, _GTAIL)]
    tb = bufb0.at[pl.ds(0, _GTAIL)]
    cp1 = pltpu.async_copy(a_hbm.at[dstb.at[pl.ds(t0, _GTAIL)]], ta, sa0)
    cp2 = pltpu.async_copy(b_hbm.at[srcb.at[pl.ds(t0, _GTAIL)]], tb, sb0)
    cp1.wait()
    cp2.wait()

    def tail_row(r, _):
        for j in range(H // 16):
            sl = pl.ds(j * 16, 16)
            bufa0[r, sl] = bufa0[r, sl] + bufb0[r, sl]
        return 0

    lax.fori_loop(0, _GTAIL, tail_row, 0, unroll=False)
    pltpu.sync_copy(ta, g_hbm.at[pl.ds(ebase + t0, _GTAIL)])


def _gather_d2_kernel(a, b, dst, src, px, py, pz):
    def body(a_hbm, b_hbm, dst_hbm, src_hbm, px_hbm, py_hbm, pz_hbm,
             g_hbm, d2_hbm, bufa0, bufb0, bufa1, bufb1, dstb, srcb,
             pxv, pyv, pzv, d2b, sa0, sb0, sa1, sb1):
        _gather_body_common(
            a_hbm, b_hbm, dst_hbm, src_hbm, g_hbm,
            bufa0, bufb0, bufa1, bufb1, dstb, srcb, sa0, sb0, sa1, sb1,
            extra=(px_hbm, py_hbm, pz_hbm, d2_hbm, pxv, pyv, pzv, d2b))

    return pl.kernel(
        body,
        out_type=[
            jax.ShapeDtypeStruct((E, H), jnp.float32),
            jax.ShapeDtypeStruct((E,), jnp.float32),
        ],
        mesh=_sc_mesh(),
        compiler_params=pltpu.CompilerParams(needs_layout_passes=False),
        scratch_types=[
            pltpu.VMEM((_GC, H), jnp.float32),
            pltpu.VMEM((_GC, H), jnp.float32),
            pltpu.VMEM((_GC, H), jnp.float32),
            pltpu.VMEM((_GC, H), jnp.float32),
            pltpu.VMEM((_EPT,), jnp.int32),
            pltpu.VMEM((_EPT,), jnp.int32),
            pltpu.VMEM((N,), jnp.float32),
            pltpu.VMEM((N,), jnp.float32),
            pltpu.VMEM((N,), jnp.float32),
            pltpu.VMEM((_EPT,), jnp.float32),
            pltpu.SemaphoreType.DMA,
            pltpu.SemaphoreType.DMA,
            pltpu.SemaphoreType.DMA,
            pltpu.SemaphoreType.DMA,
        ],
    )(a, b, dst, src, px, py, pz)


def _gather_kernel(a, b, dst, src):
    def body(a_hbm, b_hbm, dst_hbm, src_hbm, g_hbm,
             bufa0, bufb0, bufa1, bufb1, dstb, srcb, sa0, sb0, sa1, sb1):
        _gather_body_common(a_hbm, b_hbm, dst_hbm, src_hbm, g_hbm,
                            bufa0, bufb0, bufa1, bufb1, dstb, srcb,
                            sa0, sb0, sa1, sb1)

    return pl.kernel(
        body,
        out_type=jax.ShapeDtypeStruct((E, H), jnp.float32),
        mesh=_sc_mesh(),
        compiler_params=pltpu.CompilerParams(needs_layout_passes=False),
        scratch_types=[
            pltpu.VMEM((_GC, H), jnp.float32),
            pltpu.VMEM((_GC, H), jnp.float32),
            pltpu.VMEM((_GC, H), jnp.float32),
            pltpu.VMEM((_GC, H), jnp.float32),
            pltpu.VMEM((_EPT,), jnp.int32),
            pltpu.VMEM((_EPT,), jnp.int32),
            pltpu.SemaphoreType.DMA,
            pltpu.SemaphoreType.DMA,
            pltpu.SemaphoreType.DMA,
            pltpu.SemaphoreType.DMA,
        ],
    )(a, b, dst, src)


_RPT = 320
_ACCR = _RPT + 8
_LTRASH = _RPT
_CAP = 8192
_SB = 32
_SCN = 2048


def _scatter_body_common(v_hbm, dst_hbm, s_hbm, acc, vb, dstb, listE, listL,
                         sem0, sem1, cnt=None):
    wid = lax.axis_index("s") * _NC + lax.axis_index("c")
    base = wid * _RPT
    if cnt is not None:
        cnt_hbm, cntacc = cnt

    zvec = jnp.zeros((16,), jnp.float32)
    zi = jnp.zeros((16,), jnp.int32)
    ti = jnp.full((16,), _LTRASH, jnp.int32)
    ovec = jnp.ones((16,), jnp.float32)

    def zero_row(r, _):
        for j in range(H // 16):
            acc[r, pl.ds(j * 16, 16)] = zvec
        if cnt is not None:
            cntacc[r, :] = zvec
        return 0

    lax.fori_loop(0, _ACCR, zero_row, 0, unroll=False)

    def prefill(i, _):
        listE[pl.ds(i * 16, 16)] = wid * 4096 + i * 16 + lax.iota(jnp.int32, 16)
        listL[pl.ds(i * 16, 16)] = ti
        return 0

    lax.fori_loop(0, _CAP // 16, prefill, 0, unroll=False)

    def scan_chunk(ci, off):
        pltpu.sync_copy(dst_hbm.at[pl.ds(ci * _SCN, _SCN)], dstb)

        def step(j, off):
            dv = dstb[pl.ds(j * 16, 16)]
            inb = (dv >= base) & (dv < base + _RPT)
            eid = ci * _SCN + j * 16 + lax.iota(jnp.int32, 16)
            lid = dv - base

            @pl.when(off <= _CAP - 16)
            def _():
                plsc.store_compressed(listE.at[pl.ds(off, 16)], eid,
                                      mask=inb)
                plsc.store_compressed(listL.at[pl.ds(off, 16)], lid,
                                      mask=inb)

            return off + plsc.all_reduce_population_count(inb)[0]

        return lax.fori_loop(0, _SCN // 16, step, off, unroll=False)

    off = lax.fori_loop(0, E // _SCN, scan_chunk, 0, unroll=False)

    def add_row(li, r, buf):
        for j in range(H // 16):
            sl = pl.ds(j * 16, 16)
            acc[li, sl] = acc[li, sl] + buf[r, sl]
        if cnt is not None:
            cntacc[li, pl.ds(0, 16)] = cntacc[li, pl.ds(0, 16)] + ovec

    vb0 = vb.at[0]
    vb1 = vb.at[1]

    @pl.when(off <= _CAP)
    def _():
        nbat = (off + _SB - 1) // _SB

        def start_batch(b, buf, sem):
            @pl.when(b < nbat)
            def _():
                pltpu.async_copy(v_hbm.at[listE.at[pl.ds(b * _SB, _SB)]],
                                 buf, sem)

        def finish_batch(b, buf, sem):
            @pl.when(b < nbat)
            def _():
                pltpu.make_async_copy(
                    v_hbm.at[listE.at[pl.ds(b * _SB, _SB)]], buf, sem).wait()

                def addgrp(g, _):
                    lv = listL[pl.ds(b * _SB + g * 16, 16)]
                    for k in range(16):
                        add_row(lv[k], g * 16 + k, buf)
                    return 0

                lax.fori_loop(0, _SB // 16, addgrp, 0, unroll=False)

        start_batch(0, vb0, sem0)

        def pair(b2, _):
            b = 2 * b2
            start_batch(b + 1, vb1, sem1)
            finish_batch(b, vb0, sem0)
            start_batch(b + 2, vb0, sem0)
            finish_batch(b + 1, vb1, sem1)
            return 0

        lax.fori_loop(0, _CAP // _SB // 2, pair, 0, unroll=False)

    @pl.when(off > _CAP)
    def _():
        def chunk(ci, _):
            pltpu.sync_copy(dst_hbm.at[pl.ds(ci * _SB, _SB)],
                            dstb.at[pl.ds(0, _SB)])
            pltpu.sync_copy(v_hbm.at[pl.ds(ci * _SB, _SB)], vb0)

            def grp(g, _):
                dv = dstb[pl.ds(g * 16, 16)]
                lid = dv - base
                for k in range(16):
                    d = dv[k]

                    @pl.when((d >= base) & (d < base + _RPT))
                    def _():
                        add_row(lid[k], g * 16 + k, vb0)
                return 0

            lax.fori_loop(0, _SB // 16, grp, 0, unroll=False)
            return 0

        lax.fori_loop(0, E // _SB, chunk, 0, unroll=False)

    @pl.when(wid < _NW - 1)
    def _():
        pltpu.sync_copy(acc.at[pl.ds(0, _RPT)], s_hbm.at[pl.ds(base, _RPT)])
        if cnt is not None:
            pltpu.sync_copy(cntacc.at[pl.ds(0, _RPT)],
                            cnt_hbm.at[pl.ds(base, _RPT)])

    tail = N - (_NW - 1) * _RPT

    @pl.when(wid == _NW - 1)
    def _():
        pltpu.sync_copy(acc.at[pl.ds(0, tail)], s_hbm.at[pl.ds(base, tail)])
        if cnt is not None:
            pltpu.sync_copy(cntacc.at[pl.ds(0, tail)],
                            cnt_hbm.at[pl.ds(base, tail)])


def _count_kernel(dst):
    def body(dst_hbm, cnt_hbm, cntacc, dstb, listL):
        wid = lax.axis_index("s") * _NC + lax.axis_index("c")
        base = wid * _RPT
        zvec = jnp.zeros((16,), jnp.float32)
        ovec = jnp.ones((16,), jnp.float32)
        ti = jnp.full((16,), _LTRASH, jnp.int32)

        def zero_row(r, _):
            cntacc[r, pl.ds(0, 16)] = zvec
            return 0

        lax.fori_loop(0, _ACCR, zero_row, 0, unroll=False)

        def prefill(i, _):
            listL[pl.ds(i * 16, 16)] = ti
            return 0

        lax.fori_loop(0, _CAP // 16, prefill, 0, unroll=False)

        def scan_chunk(ci, off):
            pltpu.sync_copy(dst_hbm.at[pl.ds(ci * _SCN, _SCN)], dstb)

            def step(j, off):
                dv = dstb[pl.ds(j * 16, 16)]
                inb = (dv >= base) & (dv < base + _RPT)
                lid = dv - base

                @pl.when(off <= _CAP - 16)
                def _():
                    plsc.store_compressed(listL.at[pl.ds(off, 16)], lid,
                                          mask=inb)

                return off + plsc.all_reduce_population_count(inb)[0]

            return lax.fori_loop(0, _SCN // 16, step, off, unroll=False)

        off = lax.fori_loop(0, E // _SCN, scan_chunk, 0, unroll=False)

        @pl.when(off <= _CAP)
        def _():
            ngrp = (off + 15) // 16

            def grp(g, _):
                @pl.when(g < ngrp)
                def _():
                    lv = listL[pl.ds(g * 16, 16)]
                    for k in range(16):
                        li = lv[k]
                        cntacc[li, pl.ds(0, 16)] = cntacc[li, pl.ds(0, 16)] + ovec
                return 0

            lax.fori_loop(0, _CAP // 16, grp, 0, unroll=False)

        @pl.when(off > _CAP)
        def _():
            def chunk(ci, _):
                pltpu.sync_copy(dst_hbm.at[pl.ds(ci * _SCN, _SCN)], dstb)

                def sgrp(j, _):
                    dv = dstb[pl.ds(j * 16, 16)]
                    lid = dv - base
                    for k in range(16):
                        d = dv[k]

                        @pl.when((d >= base) & (d < base + _RPT))
                        def _():
                            li = lid[k]
                            cntacc[li, pl.ds(0, 16)] = cntacc[li, pl.ds(0, 16)] + ovec
                    return 0

                lax.fori_loop(0, _SCN // 16, sgrp, 0, unroll=False)
                return 0

            lax.fori_loop(0, E // _SCN, chunk, 0, unroll=False)

        @pl.when(wid < _NW - 1)
        def _():
            pltpu.sync_copy(cntacc.at[pl.ds(0, _RPT)],
                            cnt_hbm.at[pl.ds(base, _RPT)])

        tail = N - (_NW - 1) * _RPT

        @pl.when(wid == _NW - 1)
        def _():
            pltpu.sync_copy(cntacc.at[pl.ds(0, tail)],
                            cnt_hbm.at[pl.ds(base, tail)])

    return pl.kernel(
        body,
        out_type=jax.ShapeDtypeStruct((N, 16), jnp.float32),
        mesh=_sc_mesh(),
        compiler_params=pltpu.CompilerParams(needs_layout_passes=False),
        scratch_types=[
            pltpu.VMEM((_ACCR, 16), jnp.float32),
            pltpu.VMEM((_SCN,), jnp.int32),
            pltpu.VMEM((_CAP,), jnp.int32),
        ],
    )(dst)


def _scatter_kernel(v, dst):
    def body(v_hbm, dst_hbm, s_hbm, acc, vb, dstb, listE, listL, sem0,
             sem1):
        _scatter_body_common(v_hbm, dst_hbm, s_hbm, acc, vb, dstb, listE,
                             listL, sem0, sem1)

    return pl.kernel(
        body,
        out_type=jax.ShapeDtypeStruct((N, H), jnp.float32),
        mesh=_sc_mesh(),
        compiler_params=pltpu.CompilerParams(needs_layout_passes=False),
        scratch_types=[
            pltpu.VMEM((_ACCR, H), jnp.float32),
            pltpu.VMEM((2, _SB, H), jnp.float32),
            pltpu.VMEM((_SCN,), jnp.int32),
            pltpu.VMEM((_CAP,), jnp.int32),
            pltpu.VMEM((_CAP,), jnp.int32),
            pltpu.SemaphoreType.DMA,
            pltpu.SemaphoreType.DMA,
        ],
    )(v, dst)


def _conv(x, src, dst, d2, ea, cnt, pxyz, W1, b1, W2, b2, W3, b3, W4, b4,
          residual):
    first = d2 is None
    a, b = _node_proj(x, W1[:2 * D, :])
    if first:
        g, d2f = _gather_d2_kernel(a, b, dst, src, *pxyz)
        d2 = d2f.reshape(E, 1)
    else:
        g = _gather_kernel(a, b, dst, src)
    w_d = W1[2 * D:2 * D + 1, :]
    w_e = W1[2 * D + 1:, :]
    v = _edge_mlp(g, d2, ea, w_d, w_e, b1.reshape(1, H), W2,
                  b2.reshape(1, H))
    s = _scatter_kernel(v, dst)
    h = _node_mlp(x, s, cnt, W3, b3.reshape(1, H), W4, b4.reshape(1, H),
                  residual)
    return h, d2


def kernel(x, edge_index, edge_attr, pos,
           c1_W1, c1_b1, c1_W2, c1_b2, c1_W3, c1_b3, c1_W4, c1_b4,
           c2_W1, c2_b1, c2_W2, c2_b2, c2_W3, c2_b3, c2_W4, c2_b4):
    src = edge_index[0]
    dst = edge_index[1]
    pxyz = (pos[:, 0], pos[:, 1], pos[:, 2])
    cnt = _count_kernel(dst)[:, :1]
    h, d2 = _conv(x, src, dst, None, edge_attr, cnt, pxyz,
                  c1_W1, c1_b1, c1_W2, c1_b2, c1_W3, c1_b3, c1_W4,
                  c1_b4, True)
    h, _ = _conv(h, src, dst, d2, edge_attr, cnt, pxyz,
                 c2_W1, c2_b1, c2_W2, c2_b2, c2_W3, c2_b3, c2_W4,
                 c2_b4, False)
    return h


# speedup vs baseline: 2.6104x; 1.1621x over previous
import functools

import jax
import jax.numpy as jnp
from jax import lax
from jax.experimental import pallas as pl
from jax.experimental.pallas import tpu as pltpu, tpu_sc as plsc

N = 10000
E = 160000
D = 256
ED = 16
H = 256

BN = 1000
BE = 2000


def _silu(v):
    return v * jax.nn.sigmoid(v)


def _node_proj_body(x_ref, w_ref, a_ref, b_ref):
    xb = x_ref[...]
    a_ref[...] = jnp.dot(xb, w_ref[:D, :], preferred_element_type=jnp.float32)
    b_ref[...] = jnp.dot(xb, w_ref[D:, :], preferred_element_type=jnp.float32)


def _node_proj(x, w_ab):
    return pl.pallas_call(
        _node_proj_body,
        grid=(N // BN,),
        in_specs=[
            pl.BlockSpec((BN, D), lambda i: (i, 0)),
            pl.BlockSpec((2 * D, H), lambda i: (0, 0)),
        ],
        out_specs=[
            pl.BlockSpec((BN, H), lambda i: (i, 0)),
            pl.BlockSpec((BN, H), lambda i: (i, 0)),
        ],
        out_shape=[
            jax.ShapeDtypeStruct((N, H), jnp.float32),
            jax.ShapeDtypeStruct((N, H), jnp.float32),
        ],
    )(x, w_ab)


def _edge_mlp_body(g_ref, d2_ref, ea_ref, wd_ref, we_ref, b1_ref, w2_ref,
                   b2_ref, v_ref):
    u = (g_ref[...]
         + d2_ref[...] * wd_ref[...]
         + jnp.dot(ea_ref[...], we_ref[...], preferred_element_type=jnp.float32)
         + b1_ref[...])
    m1 = _silu(u)
    v = jnp.dot(m1, w2_ref[...], preferred_element_type=jnp.float32) + b2_ref[...]
    v_ref[...] = _silu(v)


def _edge_mlp(g, d2, ea, w_d, w_e, b1, w2, b2):
    return pl.pallas_call(
        _edge_mlp_body,
        grid=(E // BE,),
        in_specs=[
            pl.BlockSpec((BE, H), lambda i: (i, 0)),
            pl.BlockSpec((BE, 1), lambda i: (i, 0)),
            pl.BlockSpec((BE, ED), lambda i: (i, 0)),
            pl.BlockSpec((1, H), lambda i: (0, 0)),
            pl.BlockSpec((ED, H), lambda i: (0, 0)),
            pl.BlockSpec((1, H), lambda i: (0, 0)),
            pl.BlockSpec((H, H), lambda i: (0, 0)),
            pl.BlockSpec((1, H), lambda i: (0, 0)),
        ],
        out_specs=pl.BlockSpec((BE, H), lambda i: (i, 0)),
        out_shape=jax.ShapeDtypeStruct((E, H), jnp.float32),
    )(g, d2, ea, w_d, w_e, b1, w2, b2)


def _node_mlp_body(x_ref, s_ref, cnt_ref, w3_ref, b3_ref, w4_ref, b4_ref,
                   h_ref, *, residual):
    xb = x_ref[...]
    agg = s_ref[...] / jnp.maximum(cnt_ref[...], 1.0)
    pre = (jnp.dot(xb, w3_ref[:D, :], preferred_element_type=jnp.float32)
           + jnp.dot(agg, w3_ref[D:, :], preferred_element_type=jnp.float32)
           + b3_ref[...])
    h = jnp.dot(_silu(pre), w4_ref[...], preferred_element_type=jnp.float32) \
        + b4_ref[...]
    if residual:
        h = h + xb
    h_ref[...] = h


def _node_mlp(x, s, cnt, w3, b3, w4, b4, residual):
    return pl.pallas_call(
        functools.partial(_node_mlp_body, residual=residual),
        grid=(N // BN,),
        in_specs=[
            pl.BlockSpec((BN, D), lambda i: (i, 0)),
            pl.BlockSpec((BN, H), lambda i: (i, 0)),
            pl.BlockSpec((BN, 1), lambda i: (i, 0)),
            pl.BlockSpec((D + H, H), lambda i: (0, 0)),
            pl.BlockSpec((1, H), lambda i: (0, 0)),
            pl.BlockSpec((H, H), lambda i: (0, 0)),
            pl.BlockSpec((1, H), lambda i: (0, 0)),
        ],
        out_specs=pl.BlockSpec((BN, H), lambda i: (i, 0)),
        out_shape=jax.ShapeDtypeStruct((N, H), jnp.float32),
    )(x, s, cnt, w3, b3, w4, b4)


_NC = 2
_NS = 16
_NW = _NC * _NS
_C = 128
_NCHUNK = E // _C
_GIT = (_NCHUNK + _NW - 1) // _NW
_SIT = (_NCHUNK + _NS - 1) // _NS
_HALF = N // 2
_RPS = 320
_ACC_R = _RPS * _NS + 8
_TRASH = 5120
_TAIL = _HALF - (_NS - 1) * _RPS


def _sc_mesh():
    return plsc.VectorSubcoreMesh(core_axis_name="c", subcore_axis_name="s")


_EPT = E // _NW
_GC = 64
_GFULL = _EPT // _GC
_GTAIL = _EPT - _GFULL * _GC


def _gather_body_common(a_hbm, b_hbm, dst_hbm, src_hbm, g_hbm,
                        bufa0, bufb0, bufa1, bufb1, dstb, srcb,
                        sa0, sb0, sa1, sb1, extra=None):
    wid = lax.axis_index("s") * _NC + lax.axis_index("c")
    ebase = wid * _EPT
    pltpu.sync_copy(dst_hbm.at[pl.ds(ebase, _EPT)], dstb)
    pltpu.sync_copy(src_hbm.at[pl.ds(ebase, _EPT)], srcb)

    if extra is not None:
        px_hbm, py_hbm, pz_hbm, d2_hbm, pxv, pyv, pzv, d2b = extra
        pltpu.sync_copy(px_hbm, pxv)
        pltpu.sync_copy(py_hbm, pyv)
        pltpu.sync_copy(pz_hbm, pzv)

        def d2_step(j, _):
            sl = pl.ds(j * 16, 16)
            dv = dstb[sl]
            sv = srcb[sl]
            dx = plsc.load_gather(pxv, [dv]) - plsc.load_gather(pxv, [sv])
            dy = plsc.load_gather(pyv, [dv]) - plsc.load_gather(pyv, [sv])
            dz = plsc.load_gather(pzv, [dv]) - plsc.load_gather(pzv, [sv])
            d2b[sl] = dx * dx + dy * dy + dz * dz
            return 0

        lax.fori_loop(0, _EPT // 16, d2_step, 0, unroll=False)
        sl = pl.ds(_EPT - 16, 16)
        dv = dstb[sl]
        sv = srcb[sl]
        dx = plsc.load_gather(pxv, [dv]) - plsc.load_gather(pxv, [sv])
        dy = plsc.load_gather(pyv, [dv]) - plsc.load_gather(pyv, [sv])
        dz = plsc.load_gather(pzv, [dv]) - plsc.load_gather(pzv, [sv])
        d2b[sl] = dx * dx + dy * dy + dz * dz
        pltpu.sync_copy(d2b, d2_hbm.at[pl.ds(ebase, _EPT)])

    def start(k, ba, bb, s1, s2):
        @pl.when(k < _GFULL)
        def _():
            pltpu.async_copy(a_hbm.at[dstb.at[pl.ds(k * _GC, _GC)]], ba, s1)
            pltpu.async_copy(b_hbm.at[srcb.at[pl.ds(k * _GC, _GC)]], bb, s2)

    def finish(k, ba, bb, s1, s2):
        @pl.when(k < _GFULL)
        def _():
            pltpu.make_async_copy(
                a_hbm.at[dstb.at[pl.ds(k * _GC, _GC)]], ba, s1).wait()
            pltpu.make_async_copy(
                b_hbm.at[srcb.at[pl.ds(k * _GC, _GC)]], bb, s2).wait()

            def add_row(r, _):
                for j in range(H // 16):
                    sl = pl.ds(j * 16, 16)
                    ba[r, sl] = ba[r, sl] + bb[r, sl]
                return 0

            lax.fori_loop(0, _GC, add_row, 0, unroll=False)
            pltpu.sync_copy(ba, g_hbm.at[pl.ds(ebase + k * _GC, _GC)])

    start(0, bufa0, bufb0, sa0, sb0)

    def pair(i, _):
        k = 2 * i
        start(k + 1, bufa1, bufb1, sa1, sb1)
        finish(k, bufa0, bufb0, sa0, sb0)
        start(k + 2, bufa0, bufb0, sa0, sb0)
        finish(k + 1, bufa1, bufb1, sa1, sb1)
        return 0

    lax.fori_loop(0, (_GFULL + 1) // 2, pair, 0, unroll=False)

    t0 = _GFULL * _GC
    ta = bufa0.at[pl.ds(0, _GTAIL)]
    tb = bufb0.at[pl.ds(0, _GTAIL)]
    cp1 = pltpu.async_copy(a_hbm.at[dstb.at[pl.ds(t0, _GTAIL)]], ta, sa0)
    cp2 = pltpu.async_copy(b_hbm.at[srcb.at[pl.ds(t0, _GTAIL)]], tb, sb0)
    cp1.wait()
    cp2.wait()

    def tail_row(r, _):
        for j in range(H // 16):
            sl = pl.ds(j * 16, 16)
            bufa0[r, sl] = bufa0[r, sl] + bufb0[r, sl]
        return 0

    lax.fori_loop(0, _GTAIL, tail_row, 0, unroll=False)
    pltpu.sync_copy(ta, g_hbm.at[pl.ds(ebase + t0, _GTAIL)])


def _gather_d2_kernel(a, b, dst, src, px, py, pz):
    def body(a_hbm, b_hbm, dst_hbm, src_hbm, px_hbm, py_hbm, pz_hbm,
             g_hbm, d2_hbm, bufa0, bufb0, bufa1, bufb1, dstb, srcb,
             pxv, pyv, pzv, d2b, sa0, sb0, sa1, sb1):
        _gather_body_common(
            a_hbm, b_hbm, dst_hbm, src_hbm, g_hbm,
            bufa0, bufb0, bufa1, bufb1, dstb, srcb, sa0, sb0, sa1, sb1,
            extra=(px_hbm, py_hbm, pz_hbm, d2_hbm, pxv, pyv, pzv, d2b))

    return pl.kernel(
        body,
        out_type=[
            jax.ShapeDtypeStruct((E, H), jnp.float32),
            jax.ShapeDtypeStruct((E,), jnp.float32),
        ],
        mesh=_sc_mesh(),
        compiler_params=pltpu.CompilerParams(needs_layout_passes=False),
        scratch_types=[
            pltpu.VMEM((_GC, H), jnp.float32),
            pltpu.VMEM((_GC, H), jnp.float32),
            pltpu.VMEM((_GC, H), jnp.float32),
            pltpu.VMEM((_GC, H), jnp.float32),
            pltpu.VMEM((_EPT,), jnp.int32),
            pltpu.VMEM((_EPT,), jnp.int32),
            pltpu.VMEM((N,), jnp.float32),
            pltpu.VMEM((N,), jnp.float32),
            pltpu.VMEM((N,), jnp.float32),
            pltpu.VMEM((_EPT,), jnp.float32),
            pltpu.SemaphoreType.DMA,
            pltpu.SemaphoreType.DMA,
            pltpu.SemaphoreType.DMA,
            pltpu.SemaphoreType.DMA,
        ],
    )(a, b, dst, src, px, py, pz)


def _gather_kernel(a, b, dst, src):
    def body(a_hbm, b_hbm, dst_hbm, src_hbm, g_hbm,
             bufa0, bufb0, bufa1, bufb1, dstb, srcb, sa0, sb0, sa1, sb1):
        _gather_body_common(a_hbm, b_hbm, dst_hbm, src_hbm, g_hbm,
                            bufa0, bufb0, bufa1, bufb1, dstb, srcb,
                            sa0, sb0, sa1, sb1)

    return pl.kernel(
        body,
        out_type=jax.ShapeDtypeStruct((E, H), jnp.float32),
        mesh=_sc_mesh(),
        compiler_params=pltpu.CompilerParams(needs_layout_passes=False),
        scratch_types=[
            pltpu.VMEM((_GC, H), jnp.float32),
            pltpu.VMEM((_GC, H), jnp.float32),
            pltpu.VMEM((_GC, H), jnp.float32),
            pltpu.VMEM((_GC, H), jnp.float32),
            pltpu.VMEM((_EPT,), jnp.int32),
            pltpu.VMEM((_EPT,), jnp.int32),
            pltpu.SemaphoreType.DMA,
            pltpu.SemaphoreType.DMA,
            pltpu.SemaphoreType.DMA,
            pltpu.SemaphoreType.DMA,
        ],
    )(a, b, dst, src)


_RPT = 320
_ACCR = _RPT + 8
_LTRASH = _RPT
_CAP = 8192
_SB = 32
_SCN = 2048


def _scatter_body_common(v_hbm, dst_hbm, le_hbm, ll_hbm, off_hbm, s_hbm,
                         acc, vb, dstb, listE, listL, offb, sem0, sem1):
    wid = lax.axis_index("s") * _NC + lax.axis_index("c")
    base = wid * _RPT

    zvec = jnp.zeros((16,), jnp.float32)

    def zero_row(r, _):
        for j in range(H // 16):
            acc[r, pl.ds(j * 16, 16)] = zvec
        return 0

    lax.fori_loop(0, _ACCR, zero_row, 0, unroll=False)

    pltpu.sync_copy(le_hbm.at[pl.ds(wid * _CAP, _CAP)], listE)
    pltpu.sync_copy(ll_hbm.at[pl.ds(wid * _CAP, _CAP)], listL)
    pltpu.sync_copy(off_hbm.at[pl.ds(wid * 16, 16)], offb)
    off = offb[pl.ds(0, 16)][0]

    def add_row(li, r, buf):
        for j in range(H // 16):
            sl = pl.ds(j * 16, 16)
            acc[li, sl] = acc[li, sl] + buf[r, sl]

    vb0 = vb.at[0]
    vb1 = vb.at[1]

    @pl.when(off <= _CAP)
    def _():
        nbat = (off + _SB - 1) // _SB

        def start_batch(b, buf, sem):
            @pl.when(b < nbat)
            def _():
                pltpu.async_copy(v_hbm.at[listE.at[pl.ds(b * _SB, _SB)]],
                                 buf, sem)

        def finish_batch(b, buf, sem):
            @pl.when(b < nbat)
            def _():
                pltpu.make_async_copy(
                    v_hbm.at[listE.at[pl.ds(b * _SB, _SB)]], buf, sem).wait()

                def addgrp(g, _):
                    lv = listL[pl.ds(b * _SB + g * 16, 16)]
                    for k in range(16):
                        add_row(lv[k], g * 16 + k, buf)
                    return 0

                lax.fori_loop(0, _SB // 16, addgrp, 0, unroll=False)

        start_batch(0, vb0, sem0)

        def pair(b2, _):
            b = 2 * b2
            start_batch(b + 1, vb1, sem1)
            finish_batch(b, vb0, sem0)
            start_batch(b + 2, vb0, sem0)
            finish_batch(b + 1, vb1, sem1)
            return 0

        lax.fori_loop(0, _CAP // _SB // 2, pair, 0, unroll=False)

    @pl.when(off > _CAP)
    def _():
        def chunk(ci, _):
            pltpu.sync_copy(dst_hbm.at[pl.ds(ci * _SB, _SB)],
                            dstb.at[pl.ds(0, _SB)])
            pltpu.sync_copy(v_hbm.at[pl.ds(ci * _SB, _SB)], vb0)

            def grp(g, _):
                dv = dstb[pl.ds(g * 16, 16)]
                lid = dv - base
                for k in range(16):
                    d = dv[k]

                    @pl.when((d >= base) & (d < base + _RPT))
                    def _():
                        add_row(lid[k], g * 16 + k, vb0)
                return 0

            lax.fori_loop(0, _SB // 16, grp, 0, unroll=False)
            return 0

        lax.fori_loop(0, E // _SB, chunk, 0, unroll=False)

    @pl.when(wid < _NW - 1)
    def _():
        pltpu.sync_copy(acc.at[pl.ds(0, _RPT)], s_hbm.at[pl.ds(base, _RPT)])

    tail = N - (_NW - 1) * _RPT

    @pl.when(wid == _NW - 1)
    def _():
        pltpu.sync_copy(acc.at[pl.ds(0, tail)], s_hbm.at[pl.ds(base, tail)])


def _plan_kernel(dst):
    def body(dst_hbm, cnt_hbm, le_hbm, ll_hbm, off_hbm, cntacc, dstb, listE,
             listL, offb):
        wid = lax.axis_index("s") * _NC + lax.axis_index("c")
        base = wid * _RPT
        zvec = jnp.zeros((16,), jnp.float32)
        ovec = jnp.ones((16,), jnp.float32)
        ti = jnp.full((16,), _LTRASH, jnp.int32)

        def zero_row(r, _):
            cntacc[r, pl.ds(0, 16)] = zvec
            return 0

        lax.fori_loop(0, _ACCR, zero_row, 0, unroll=False)

        def prefill(i, _):
            listE[pl.ds(i * 16, 16)] = (wid * 4096 + i * 16
                                        + lax.iota(jnp.int32, 16))
            listL[pl.ds(i * 16, 16)] = ti
            return 0

        lax.fori_loop(0, _CAP // 16, prefill, 0, unroll=False)

        def scan_chunk(ci, off):
            pltpu.sync_copy(dst_hbm.at[pl.ds(ci * _SCN, _SCN)], dstb)

            def step(j, off):
                dv = dstb[pl.ds(j * 16, 16)]
                inb = (dv >= base) & (dv < base + _RPT)
                eid = ci * _SCN + j * 16 + lax.iota(jnp.int32, 16)
                lid = dv - base

                @pl.when(off <= _CAP - 16)
                def _():
                    plsc.store_compressed(listE.at[pl.ds(off, 16)], eid,
                                          mask=inb)
                    plsc.store_compressed(listL.at[pl.ds(off, 16)], lid,
                                          mask=inb)

                return off + plsc.all_reduce_population_count(inb)[0]

            return lax.fori_loop(0, _SCN // 16, step, off, unroll=False)

        off = lax.fori_loop(0, E // _SCN, scan_chunk, 0, unroll=False)

        @pl.when(off <= _CAP)
        def _():
            ngrp = (off + 15) // 16

            def grp(g, _):
                @pl.when(g < ngrp)
                def _():
                    lv = listL[pl.ds(g * 16, 16)]
                    for k in range(16):
                        li = lv[k]
                        cntacc[li, pl.ds(0, 16)] = (cntacc[li, pl.ds(0, 16)]
                                                    + ovec)
                return 0

            lax.fori_loop(0, _CAP // 16, grp, 0, unroll=False)

        @pl.when(off > _CAP)
        def _():
            def chunk(ci, _):
                pltpu.sync_copy(dst_hbm.at[pl.ds(ci * _SCN, _SCN)], dstb)

                def sgrp(j, _):
                    dv = dstb[pl.ds(j * 16, 16)]
                    lid = dv - base
                    for k in range(16):
                        d = dv[k]

                        @pl.when((d >= base) & (d < base + _RPT))
                        def _():
                            li = lid[k]
                            cntacc[li, pl.ds(0, 16)] = (
                                cntacc[li, pl.ds(0, 16)] + ovec)
                    return 0

                lax.fori_loop(0, _SCN // 16, sgrp, 0, unroll=False)
                return 0

            lax.fori_loop(0, E // _SCN, chunk, 0, unroll=False)

        pltpu.sync_copy(listE, le_hbm.at[pl.ds(wid * _CAP, _CAP)])
        pltpu.sync_copy(listL, ll_hbm.at[pl.ds(wid * _CAP, _CAP)])
        offb[pl.ds(0, 16)] = jnp.broadcast_to(off, (16,))
        pltpu.sync_copy(offb, off_hbm.at[pl.ds(wid * 16, 16)])

        @pl.when(wid < _NW - 1)
        def _():
            pltpu.sync_copy(cntacc.at[pl.ds(0, _RPT)],
                            cnt_hbm.at[pl.ds(base, _RPT)])

        tail = N - (_NW - 1) * _RPT

        @pl.when(wid == _NW - 1)
        def _():
            pltpu.sync_copy(cntacc.at[pl.ds(0, tail)],
                            cnt_hbm.at[pl.ds(base, tail)])

    return pl.kernel(
        body,
        out_type=[
            jax.ShapeDtypeStruct((N, 16), jnp.float32),
            jax.ShapeDtypeStruct((_NW * _CAP,), jnp.int32),
            jax.ShapeDtypeStruct((_NW * _CAP,), jnp.int32),
            jax.ShapeDtypeStruct((_NW * 16,), jnp.int32),
        ],
        mesh=_sc_mesh(),
        compiler_params=pltpu.CompilerParams(needs_layout_passes=False),
        scratch_types=[
            pltpu.VMEM((_ACCR, 16), jnp.float32),
            pltpu.VMEM((_SCN,), jnp.int32),
            pltpu.VMEM((_CAP,), jnp.int32),
            pltpu.VMEM((_CAP,), jnp.int32),
            pltpu.VMEM((16,), jnp.int32),
        ],
    )(dst)


def _scatter_kernel(v, dst, le, ll, offs):
    def body(v_hbm, dst_hbm, le_hbm, ll_hbm, off_hbm, s_hbm, acc, vb, dstb,
             listE, listL, offb, sem0, sem1):
        _scatter_body_common(v_hbm, dst_hbm, le_hbm, ll_hbm, off_hbm, s_hbm,
                             acc, vb, dstb, listE, listL, offb, sem0, sem1)

    return pl.kernel(
        body,
        out_type=jax.ShapeDtypeStruct((N, H), jnp.float32),
        mesh=_sc_mesh(),
        compiler_params=pltpu.CompilerParams(needs_layout_passes=False),
        scratch_types=[
            pltpu.VMEM((_ACCR, H), jnp.float32),
            pltpu.VMEM((2, _SB, H), jnp.float32),
            pltpu.VMEM((_SCN,), jnp.int32),
            pltpu.VMEM((_CAP,), jnp.int32),
            pltpu.VMEM((_CAP,), jnp.int32),
            pltpu.VMEM((16,), jnp.int32),
            pltpu.SemaphoreType.DMA,
            pltpu.SemaphoreType.DMA,
        ],
    )(v, dst, le, ll, offs)


def _conv(x, src, dst, d2, ea, cnt, plan, pxyz, W1, b1, W2, b2, W3, b3, W4,
          b4, residual):
    first = d2 is None
    a, b = _node_proj(x, W1[:2 * D, :])
    if first:
        g, d2f = _gather_d2_kernel(a, b, dst, src, *pxyz)
        d2 = d2f.reshape(E, 1)
    else:
        g = _gather_kernel(a, b, dst, src)
    w_d = W1[2 * D:2 * D + 1, :]
    w_e = W1[2 * D + 1:, :]
    v = _edge_mlp(g, d2, ea, w_d, w_e, b1.reshape(1, H), W2,
                  b2.reshape(1, H))
    s = _scatter_kernel(v, dst, *plan)
    h = _node_mlp(x, s, cnt, W3, b3.reshape(1, H), W4, b4.reshape(1, H),
                  residual)
    return h, d2


def kernel(x, edge_index, edge_attr, pos,
           c1_W1, c1_b1, c1_W2, c1_b2, c1_W3, c1_b3, c1_W4, c1_b4,
           c2_W1, c2_b1, c2_W2, c2_b2, c2_W3, c2_b3, c2_W4, c2_b4):
    src = edge_index[0]
    dst = edge_index[1]
    pxyz = (pos[:, 0], pos[:, 1], pos[:, 2])
    cnt16, le, ll, offs = _plan_kernel(dst)
    cnt = cnt16[:, :1]
    plan = (le, ll, offs)
    h, d2 = _conv(x, src, dst, None, edge_attr, cnt, plan, pxyz,
                  c1_W1, c1_b1, c1_W2, c1_b2, c1_W3, c1_b3, c1_W4,
                  c1_b4, True)
    h, _ = _conv(h, src, dst, d2, edge_attr, cnt, plan, pxyz,
                 c2_W1, c2_b1, c2_W2, c2_b2, c2_W3, c2_b3, c2_W4,
                 c2_b4, False)
    return h
